# baseline (speedup 1.0000x reference)
import functools

import jax
import jax.numpy as jnp
from jax import lax
from jax.experimental import pallas as pl
from jax.experimental.pallas import tpu as pltpu

Q, D, H, E = 32, 64, 256, 8
TOPK = 5
BLK = 65536
NEG = float("-inf")
IMAX = jnp.iinfo(jnp.int32).max


def _layer_norm(x, g, b, eps=1e-5):
    mu = jnp.mean(x, axis=-1, keepdims=True)
    var = jnp.mean((x - mu) ** 2, axis=-1, keepdims=True)
    return (x - mu) / jnp.sqrt(var + eps) * g + b


def _topk_rounds(vals, idxs, k):
    out_v, out_i = [], []
    for _ in range(k):
        m = jnp.max(vals, axis=1, keepdims=True)
        am = jnp.min(jnp.where(vals == m, idxs, IMAX), axis=1, keepdims=True)
        out_v.append(m)
        out_i.append(am)
        vals = jnp.where(idxs == am, NEG, vals)
    return jnp.concatenate(out_v, axis=1), jnp.concatenate(out_i, axis=1)


def _body(q_ref, kt_ref, w1_ref, b1_ref, g1_ref, be1_ref, w2_ref, b2_ref,
          g2_ref, be2_ref, wg1_ref, bg1_ref, wg2_ref, bg2_ref,
          tv_ref, ti_ref, gate_ref, qn_ref, cv_ref, ci_ref, *, nsteps, n_keys):
    step = pl.program_id(0)

    bdot = lambda a, b: jnp.dot(a.astype(jnp.bfloat16), b.astype(jnp.bfloat16),
                                preferred_element_type=jnp.float32)

    @pl.when(step == 0)
    def _init():
        q = q_ref[...]
        h = _layer_norm(bdot(q, w1_ref[...]) + b1_ref[...],
                        g1_ref[...], be1_ref[...])
        h = jnp.maximum(h, 0.0)
        q_emb = _layer_norm(bdot(h, w2_ref[...]) + b2_ref[...],
                            g2_ref[...], be2_ref[...])
        qn_ref[...] = (q_emb / (jnp.sqrt(jnp.sum(q_emb * q_emb, axis=-1,
                                                 keepdims=True)) + 1e-8)
                       ).astype(jnp.bfloat16)
        z = jnp.maximum(bdot(q, wg1_ref[...]) + bg1_ref[...], 0.0)
        z = bdot(z, wg2_ref[...]) + bg2_ref[...]
        z = z - jnp.max(z, axis=-1, keepdims=True)
        ez = jnp.exp(z)
        gate_ref[...] = ez / jnp.sum(ez, axis=-1, keepdims=True)
        tv_ref[...] = jnp.full((Q, TOPK), NEG, dtype=jnp.float32)
        ti_ref[...] = -1 - lax.broadcasted_iota(jnp.int32, (Q, TOPK), 1)

    kb = kt_ref[...]
    nsq = jnp.sum(kb * kb, axis=0, keepdims=True)
    inv = 1.0 / (jnp.sqrt(nsq) + 1e-8)
    kn = (kb * inv).astype(jnp.bfloat16)
    sims = jnp.dot(qn_ref[...], kn,
                   preferred_element_type=jnp.float32)
    iota_l = lax.broadcasted_iota(jnp.int32, (Q, BLK), 1)
    iota_row = lax.broadcasted_iota(jnp.int32, (1, BLK), 1)
    s = jnp.where(iota_row < (n_keys - step * BLK), sims, NEG)

    fv, fi = s, iota_l
    width = BLK
    for _ in range(3):
        width //= 2
        va, vb = fv[:, :width], fv[:, width:]
        ia, ib = fi[:, :width], fi[:, width:]
        take = va >= vb
        fv = jnp.maximum(va, vb)
        fi = jnp.where(take, ia, ib)
    cand_v, cand_li = _topk_rounds(fv, fi, TOPK)
    cv_ref[...] = cand_v
    ci_ref[...] = cand_li

    t5 = cand_v[:, TOPK - 1:TOPK]
    cnt = jnp.sum(jnp.where(s >= t5, 1.0, 0.0), axis=1, keepdims=True)

    @pl.when(jnp.max(cnt) > float(TOPK))
    def _exact_fallback():
        full_v, full_li = _topk_rounds(s, iota_l, TOPK)
        cv_ref[...] = full_v
        ci_ref[...] = full_li

    comb_v = jnp.concatenate([tv_ref[...], cv_ref[...]], axis=1)
    comb_i = jnp.concatenate([ti_ref[...], ci_ref[...] + step * BLK], axis=1)
    new_v, new_i = _topk_rounds(comb_v, comb_i, TOPK)
    tv_ref[...] = new_v
    ti_ref[...] = new_i


def kernel(queries, keys, W1, b1, g1, be1, W2, b2, g2, be2, Wg1, bg1, Wg2, bg2):
    n_keys = keys.shape[0]
    nsteps = pl.cdiv(n_keys, BLK)
    row = lambda v: v.reshape(1, -1)
    keys_t = jnp.swapaxes(keys, 0, 1)

    full = lambda shape: pl.BlockSpec(shape, lambda i: (0, 0))
    grid_spec = pltpu.PrefetchScalarGridSpec(
        num_scalar_prefetch=0,
        grid=(nsteps,),
        in_specs=[
            full((Q, D)),
            pl.BlockSpec((D, BLK), lambda i: (0, i)),
            full((D, D)), full((1, D)), full((1, D)), full((1, D)),
            full((D, D)), full((1, D)), full((1, D)), full((1, D)),
            full((D, H)), full((1, H)),
            full((H, E)), full((1, E)),
        ],
        out_specs=[full((Q, TOPK)), full((Q, TOPK)), full((Q, E))],
        scratch_shapes=[pltpu.VMEM((Q, D), jnp.bfloat16),
                        pltpu.VMEM((Q, TOPK), jnp.float32),
                        pltpu.VMEM((Q, TOPK), jnp.int32)],
    )
    out_shapes = [
        jax.ShapeDtypeStruct((Q, TOPK), jnp.float32),
        jax.ShapeDtypeStruct((Q, TOPK), jnp.int32),
        jax.ShapeDtypeStruct((Q, E), jnp.float32),
    ]
    tv, ti, gate = pl.pallas_call(
        functools.partial(_body, nsteps=nsteps, n_keys=n_keys),
        grid_spec=grid_spec,
        out_shape=out_shapes,
        compiler_params=pltpu.CompilerParams(
            dimension_semantics=("arbitrary",),
        ),
    )(queries, keys_t, W1, row(b1), row(g1), row(be1), W2, row(b2), row(g2),
      row(be2), Wg1, row(bg1), Wg2, row(bg2))
    return tv, ti, gate


# speedup vs baseline: 1.0309x; 1.0309x over previous
import functools

import jax
import jax.numpy as jnp
from jax import lax
from jax.experimental import pallas as pl
from jax.experimental.pallas import tpu as pltpu

Q, D, H, E = 32, 64, 256, 8
TOPK = 5
BLK = 65536
NEG = float("-inf")
IMAX = jnp.iinfo(jnp.int32).max


def _layer_norm(x, g, b, eps=1e-5):
    mu = jnp.mean(x, axis=-1, keepdims=True)
    var = jnp.mean((x - mu) ** 2, axis=-1, keepdims=True)
    return (x - mu) / jnp.sqrt(var + eps) * g + b


def _topk_rounds(vals, idxs, k):
    out_v, out_i = [], []
    for _ in range(k):
        m = jnp.max(vals, axis=1, keepdims=True)
        am = jnp.min(jnp.where(vals == m, idxs, IMAX), axis=1, keepdims=True)
        out_v.append(m)
        out_i.append(am)
        vals = jnp.where(idxs == am, NEG, vals)
    return (jnp.concatenate(out_v, axis=1), jnp.concatenate(out_i, axis=1),
            vals)


def _body(q_ref, kt_ref, w1_ref, b1_ref, g1_ref, be1_ref, w2_ref, b2_ref,
          g2_ref, be2_ref, wg1_ref, bg1_ref, wg2_ref, bg2_ref,
          tv_ref, ti_ref, gate_ref, qn_ref, cv_ref, ci_ref, *, nsteps, n_keys):
    step = pl.program_id(0)

    bdot = lambda a, b: jnp.dot(a.astype(jnp.bfloat16), b.astype(jnp.bfloat16),
                                preferred_element_type=jnp.float32)

    @pl.when(step == 0)
    def _init():
        q = q_ref[...]
        h = _layer_norm(bdot(q, w1_ref[...]) + b1_ref[...],
                        g1_ref[...], be1_ref[...])
        h = jnp.maximum(h, 0.0)
        q_emb = _layer_norm(bdot(h, w2_ref[...]) + b2_ref[...],
                            g2_ref[...], be2_ref[...])
        qn_ref[...] = (q_emb / (jnp.sqrt(jnp.sum(q_emb * q_emb, axis=-1,
                                                 keepdims=True)) + 1e-8)
                       ).astype(jnp.bfloat16)
        z = jnp.maximum(bdot(q, wg1_ref[...]) + bg1_ref[...], 0.0)
        z = bdot(z, wg2_ref[...]) + bg2_ref[...]
        z = z - jnp.max(z, axis=-1, keepdims=True)
        ez = jnp.exp(z)
        gate_ref[...] = ez / jnp.sum(ez, axis=-1, keepdims=True)
        tv_ref[...] = jnp.full((Q, TOPK), NEG, dtype=jnp.float32)
        ti_ref[...] = -1 - lax.broadcasted_iota(jnp.int32, (Q, TOPK), 1)

    kb = kt_ref[...]
    nsq = jnp.sum(kb * kb, axis=0, keepdims=True)
    inv = 1.0 / (jnp.sqrt(nsq) + 1e-8)
    kn = (kb * inv).astype(jnp.bfloat16)
    sims = jnp.dot(qn_ref[...], kn,
                   preferred_element_type=jnp.float32)
    iota_l = lax.broadcasted_iota(jnp.int32, (Q, BLK), 1)
    iota_row = lax.broadcasted_iota(jnp.int32, (1, BLK), 1)
    s = jnp.where(iota_row < (n_keys - step * BLK), sims, NEG)

    fv, fi = s, iota_l
    lmax = None
    width = BLK
    for _ in range(3):
        width //= 2
        va, vb = fv[:, :width], fv[:, width:]
        ia, ib = fi[:, :width], fi[:, width:]
        take = va >= vb
        lose = jnp.minimum(va, vb)
        lmax = lose if lmax is None else jnp.maximum(
            jnp.maximum(lmax[:, :width], lmax[:, width:]), lose)
        fv = jnp.maximum(va, vb)
        fi = jnp.where(take, ia, ib)
    cand_v, cand_li, resid = _topk_rounds(fv, fi, TOPK)
    cv_ref[...] = cand_v
    ci_ref[...] = cand_li

    t5 = cand_v[:, TOPK - 1:TOPK]
    sixth = jnp.maximum(jnp.max(lmax, axis=1, keepdims=True),
                        jnp.max(resid, axis=1, keepdims=True))

    @pl.when(jnp.max(jnp.where(sixth >= t5, 1.0, 0.0)) > 0.0)
    def _exact_fallback():
        full_v, full_li, _ = _topk_rounds(s, iota_l, TOPK)
        cv_ref[...] = full_v
        ci_ref[...] = full_li

    comb_v = jnp.concatenate([tv_ref[...], cv_ref[...]], axis=1)
    comb_i = jnp.concatenate([ti_ref[...], ci_ref[...] + step * BLK], axis=1)
    new_v, new_i, _ = _topk_rounds(comb_v, comb_i, TOPK)
    tv_ref[...] = new_v
    ti_ref[...] = new_i


def kernel(queries, keys, W1, b1, g1, be1, W2, b2, g2, be2, Wg1, bg1, Wg2, bg2):
    n_keys = keys.shape[0]
    nsteps = pl.cdiv(n_keys, BLK)
    row = lambda v: v.reshape(1, -1)
    keys_t = jnp.swapaxes(keys, 0, 1)

    full = lambda shape: pl.BlockSpec(shape, lambda i: (0, 0))
    grid_spec = pltpu.PrefetchScalarGridSpec(
        num_scalar_prefetch=0,
        grid=(nsteps,),
        in_specs=[
            full((Q, D)),
            pl.BlockSpec((D, BLK), lambda i: (0, i)),
            full((D, D)), full((1, D)), full((1, D)), full((1, D)),
            full((D, D)), full((1, D)), full((1, D)), full((1, D)),
            full((D, H)), full((1, H)),
            full((H, E)), full((1, E)),
        ],
        out_specs=[full((Q, TOPK)), full((Q, TOPK)), full((Q, E))],
        scratch_shapes=[pltpu.VMEM((Q, D), jnp.bfloat16),
                        pltpu.VMEM((Q, TOPK), jnp.float32),
                        pltpu.VMEM((Q, TOPK), jnp.int32)],
    )
    out_shapes = [
        jax.ShapeDtypeStruct((Q, TOPK), jnp.float32),
        jax.ShapeDtypeStruct((Q, TOPK), jnp.int32),
        jax.ShapeDtypeStruct((Q, E), jnp.float32),
    ]
    tv, ti, gate = pl.pallas_call(
        functools.partial(_body, nsteps=nsteps, n_keys=n_keys),
        grid_spec=grid_spec,
        out_shape=out_shapes,
        compiler_params=pltpu.CompilerParams(
            dimension_semantics=("arbitrary",),
        ),
    )(queries, keys_t, W1, row(b1), row(g1), row(be1), W2, row(b2), row(g2),
      row(be2), Wg1, row(bg1), Wg2, row(bg2))
    return tv, ti, gate
